# baseline (speedup 1.0000x reference)
import functools

import jax
import jax.numpy as jnp
from jax import lax
from jax.experimental import pallas as pl
from jax.experimental.pallas import tpu as pltpu
from jax.experimental.pallas import tpu_sc as plsc

_L = 16
_NW = 32
_IDX_CHUNK = 128
_WIDEN_BLK = 2000


def _widen_body(c_ref, o_ref):
    x = c_ref[...]
    o_ref[...] = jnp.concatenate([x, jnp.zeros_like(x)], axis=1)


def _widen(centers):
    n, d = centers.shape
    return pl.pallas_call(
        _widen_body,
        grid=(n // _WIDEN_BLK,),
        in_specs=[pl.BlockSpec((_WIDEN_BLK, d), lambda i: (i, 0))],
        out_specs=pl.BlockSpec((_WIDEN_BLK, 2 * d), lambda i: (i, 0)),
        out_shape=jax.ShapeDtypeStruct((n, 2 * d), jnp.float32),
    )(centers)


def _make_sc_loss(B, D):
    b_per_w = B // _NW
    n_chunk = b_per_w // _IDX_CHUNK
    half = b_per_w // 2
    mesh = plsc.VectorSubcoreMesh(core_axis_name="c", subcore_axis_name="s")

    @functools.partial(
        pl.kernel,
        mesh=mesh,
        out_type=jax.ShapeDtypeStruct((_NW, _L), jnp.float32),
        scratch_types=[
            pltpu.VMEM((b_per_w,), jnp.int32),
            pltpu.VMEM((n_chunk, _IDX_CHUNK), jnp.int32),
            pltpu.VMEM((b_per_w, 2 * D), jnp.float32),
            pltpu.VMEM((half, D), jnp.float32),
            pltpu.VMEM((_L,), jnp.float32),
            pltpu.SemaphoreType.DMA,
        ],
    )
    def sc_loss(tgt_hbm, emb_hbm, cent_hbm, out_hbm, idx_v, pidx_v, prow_v,
                emb_v, acc_v, sem):
        wid = lax.axis_index("s") * 2 + lax.axis_index("c")
        base = wid * b_per_w
        pltpu.sync_copy(tgt_hbm.at[pl.ds(base, b_per_w)], idx_v)

        def stage(c, carry):
            pidx_v[c // 8, pl.ds((c % 8) * _L, _L)] = idx_v[pl.ds(c * _L, _L)]
            return carry

        lax.fori_loop(0, b_per_w // _L, stage, 0)
        copies = [
            pltpu.async_copy(
                cent_hbm.at[pidx_v.at[g]],
                prow_v.at[pl.ds(g * _IDX_CHUNK, _IDX_CHUNK)],
                sem,
            )
            for g in range(n_chunk)
        ]

        zero = jnp.zeros((_L,), jnp.float32)
        vecs = D // _L
        accs = (zero,) * vecs

        for ph in range(2):
            pltpu.sync_copy(emb_hbm.at[pl.ds(base + ph * half, half)], emb_v)
            for g in range(n_chunk // 2):
                copies[ph * (n_chunk // 2) + g].wait()
            off = ph * half

            def body(i, accs, off=off):
                out = []
                for j in range(vecs):
                    e = emb_v[i, pl.ds(j * _L, _L)]
                    c = prow_v[off + i, pl.ds(j * _L, _L)]
                    d = e - c
                    out.append(accs[j] + d * d)
                return tuple(out)

            accs = lax.fori_loop(0, half, body, accs)

        total = accs[0]
        for j in range(1, vecs):
            total = total + accs[j]
        acc_v[...] = total
        pltpu.sync_copy(acc_v, out_hbm.at[wid])

    return sc_loss


def kernel(target, vector_embedding, centers):
    B, D = vector_embedding.shape
    tgt = target.astype(jnp.int32)
    cent_wide = _widen(centers)
    partials = _make_sc_loss(B, D)(tgt, vector_embedding, cent_wide)
    return jnp.sum(partials) * (0.5 / B)


# speedup vs baseline: 1.3876x; 1.3876x over previous
import functools

import jax
import jax.numpy as jnp
from jax import lax
from jax.experimental import pallas as pl
from jax.experimental.pallas import tpu as pltpu
from jax.experimental.pallas import tpu_sc as plsc

_L = 16
_NW = 32
_IDX_CHUNK = 128


def _make_sc_loss(B, D):
    b_per_w = B // _NW
    n_chunk = b_per_w // _IDX_CHUNK
    half = b_per_w // 2
    mesh = plsc.VectorSubcoreMesh(core_axis_name="c", subcore_axis_name="s")

    @functools.partial(
        pl.kernel,
        mesh=mesh,
        out_type=jax.ShapeDtypeStruct((_NW, _L), jnp.float32),
        scratch_types=[
            pltpu.VMEM((b_per_w,), jnp.int32),
            pltpu.VMEM((n_chunk, _IDX_CHUNK), jnp.int32),
            pltpu.VMEM((b_per_w, 2 * D), jnp.float32),
            pltpu.VMEM((half, D), jnp.float32),
            pltpu.VMEM((_L,), jnp.float32),
            pltpu.SemaphoreType.DMA,
        ],
    )
    def sc_loss(tgt_hbm, emb_hbm, cent_hbm, out_hbm, idx_v, pidx_v, prow_v,
                emb_v, acc_v, sem):
        wid = lax.axis_index("s") * 2 + lax.axis_index("c")
        base = wid * b_per_w
        pltpu.sync_copy(tgt_hbm.at[pl.ds(base, b_per_w)], idx_v)

        def stage(c, carry):
            pidx_v[c // 8, pl.ds((c % 8) * _L, _L)] = idx_v[pl.ds(c * _L, _L)]
            return carry

        lax.fori_loop(0, b_per_w // _L, stage, 0)
        copies = [
            pltpu.async_copy(
                cent_hbm.at[pidx_v.at[g]],
                prow_v.at[pl.ds(g * _IDX_CHUNK, _IDX_CHUNK)],
                sem,
            )
            for g in range(n_chunk)
        ]

        zero = jnp.zeros((_L,), jnp.float32)
        vecs = D // _L
        accs = (zero,) * vecs

        for ph in range(2):
            pltpu.sync_copy(emb_hbm.at[pl.ds(base + ph * half, half)], emb_v)
            for g in range(n_chunk // 2):
                copies[ph * (n_chunk // 2) + g].wait()
            off = ph * half

            def body(i, accs, off=off):
                out = []
                for j in range(vecs):
                    e = emb_v[i, pl.ds(j * _L, _L)]
                    c = prow_v[off + i, pl.ds(j * _L, _L)]
                    d = e - c
                    out.append(accs[j] + d * d)
                return tuple(out)

            accs = lax.fori_loop(0, half, body, accs)

        total = accs[0]
        for j in range(1, vecs):
            total = total + accs[j]
        acc_v[...] = total
        pltpu.sync_copy(acc_v, out_hbm.at[wid])

    return sc_loss


def kernel(target, vector_embedding, centers):
    B, D = vector_embedding.shape
    tgt = target.astype(jnp.int32)
    cent_wide = jnp.pad(centers, ((0, 0), (0, D)))
    partials = _make_sc_loss(B, D)(tgt, vector_embedding, cent_wide)
    return jnp.sum(partials) * (0.5 / B)
